# Initial kernel scaffold; baseline (speedup 1.0000x reference)
import jax
import jax.numpy as jnp
from jax.experimental import pallas as pl


def kernel(x, edge_index, W1, b1, W2, b2):
    raise NotImplementedError("write your pallas kernel here")


# speedup vs baseline: 178.0515x; 178.0515x over previous
import functools
import jax
import jax.numpy as jnp
from jax import lax
from jax.experimental import pallas as pl
from jax.experimental.pallas import tpu as pltpu
from jax.experimental.pallas import tpu_sc as plsc

N_NODES = 100000
LANES = 128
ROWS = 782
N_PAD = ROWS * LANES
NC = 2
NS = 16
NW = NC * NS
CHUNK = 2048
RPS = N_PAD // NS


def _m8(v):
    return pl.multiple_of(v, 8)


def _deg_body(n_chunks, per_tile, dst_hbm, ones_hbm, zeros_hbm, out_hbm,
              idx_v, ones_v, stage_v, deg_sh, sem):
    c = lax.axis_index("c")
    s = lax.axis_index("s")
    wid = c * NS + s
    pltpu.sync_copy(zeros_hbm.at[pl.ds(_m8(s * RPS), RPS)], stage_v)
    pltpu.sync_copy(stage_v, deg_sh.at[pl.ds(_m8(s * RPS), RPS)])
    pltpu.sync_copy(ones_hbm, ones_v)
    plsc.subcore_barrier()
    base = wid * per_tile

    def body(i, carry):
        off = _m8(base + i * CHUNK)
        pltpu.sync_copy(dst_hbm.at[pl.ds(off, CHUNK)], idx_v)
        pltpu.sync_copy(ones_v, deg_sh.at[idx_v], add=True)
        return carry

    lax.fori_loop(0, n_chunks, body, 0)
    plsc.subcore_barrier()
    pltpu.sync_copy(deg_sh.at[pl.ds(_m8(s * RPS), RPS)], stage_v)
    pltpu.sync_copy(stage_v,
                    out_hbm.at[pl.ds(_m8(c * N_PAD + s * RPS), RPS)])


def _edge_body(nch, n_chunks, per_tile, *refs):
    src_hbm, dst_hbm = refs[0], refs[1]
    tabs_hbm = refs[2:2 + nch]
    zeros_hbm = refs[2 + nch]
    outs_hbm = refs[3 + nch:3 + 2 * nch]
    sc = refs[3 + 2 * nch:]
    isrc_v, idst_v, stage_v = sc[0], sc[1], sc[2]
    vals_v = sc[3:3 + nch]
    tabs_sh = sc[3 + nch:3 + 2 * nch]
    aggs_sh = sc[3 + 2 * nch:3 + 3 * nch]
    sem = sc[3 + 3 * nch]

    c = lax.axis_index("c")
    s = lax.axis_index("s")
    wid = c * NS + s
    sl = pl.ds(_m8(s * RPS), RPS)
    for k in range(nch):
        pltpu.sync_copy(tabs_hbm[k].at[sl], stage_v)
        pltpu.sync_copy(stage_v, tabs_sh[k].at[sl])
    pltpu.sync_copy(zeros_hbm.at[sl], stage_v)
    for k in range(nch):
        pltpu.sync_copy(stage_v, aggs_sh[k].at[sl])
    plsc.subcore_barrier()
    base = wid * per_tile

    def body(i, carry):
        off = _m8(base + i * CHUNK)
        pltpu.sync_copy(src_hbm.at[pl.ds(off, CHUNK)], isrc_v)
        pltpu.sync_copy(dst_hbm.at[pl.ds(off, CHUNK)], idst_v)
        cps = [pltpu.async_copy(tabs_sh[k].at[isrc_v], vals_v[k], sem)
               for k in range(nch)]
        for cp in cps:
            cp.wait()
        for k in range(nch):
            pltpu.sync_copy(vals_v[k], aggs_sh[k].at[idst_v], add=True)
        return carry

    lax.fori_loop(0, n_chunks, body, 0)
    plsc.subcore_barrier()
    osl = pl.ds(_m8(c * N_PAD + s * RPS), RPS)
    for k in range(nch):
        pltpu.sync_copy(aggs_sh[k].at[sl], stage_v)
        pltpu.sync_copy(stage_v, outs_hbm[k].at[osl])


def _sc_mesh():
    return plsc.VectorSubcoreMesh(core_axis_name="c", subcore_axis_name="s")


def _edge_pass(nch, n_chunks, per_tile, src, dst, tabs, zeros):
    fn = pl.kernel(
        functools.partial(_edge_body, nch, n_chunks, per_tile),
        out_type=tuple(jax.ShapeDtypeStruct((NC * N_PAD,), jnp.float32)
                       for _ in range(nch)),
        mesh=_sc_mesh(),
        scratch_types=(
            [pltpu.VMEM((CHUNK,), jnp.int32)] * 2
            + [pltpu.VMEM((RPS,), jnp.float32)]
            + [pltpu.VMEM((CHUNK,), jnp.float32)] * nch
            + [pltpu.VMEM_SHARED((N_PAD,), jnp.float32)] * (2 * nch)
            + [pltpu.SemaphoreType.DMA]
        ),
    )
    return fn(src, dst, *tabs, zeros)


def _tc_a(degp_ref, xt_ref, dis_ref, yt_ref):
    deg = degp_ref[0] + degp_ref[1] + 1.0
    dis = lax.rsqrt(deg)
    dis_ref[...] = dis
    for ch in range(3):
        yt_ref[ch] = xt_ref[ch] * dis


def _tc_b(a0_ref, a1_ref, a2_ref, xt_ref, dis_ref, w1_ref, b1_ref, w2_ref,
          z_ref, q_ref):
    dis = dis_ref[...]
    w1 = w1_ref[...]
    b1 = b1_ref[...]
    w2 = w2_ref[...]
    aggs = [a0_ref, a1_ref, a2_ref]
    svec = []
    for ch in range(3):
        agg = aggs[ch][0] + aggs[ch][1]
        svec.append(dis * (agg + dis * xt_ref[ch]))
    q = jnp.zeros_like(dis)
    for j in range(16):
        h = svec[0] * w1[0, j] + svec[1] * w1[1, j] + svec[2] * w1[2, j] + b1[j]
        h = jnp.where(h > 0, h, jnp.exp(h) - 1.0)
        q = q + h * w2[j, 0]
    q_ref[...] = q
    z_ref[...] = q * dis


def _tc_c(aggp_ref, dis_ref, q_ref, b2_ref, out_ref):
    dis = dis_ref[...]
    agg = aggp_ref[0] + aggp_ref[1]
    o = dis * (agg + dis * q_ref[...]) + b2_ref[0]
    out_ref[...] = jnp.where(o > 0, o, jnp.exp(o) - 1.0)


def kernel(x, edge_index, W1, b1, W2, b2):
    E = edge_index.shape[1]
    grain = NW * CHUNK
    E_pad = ((E + grain - 1) // grain) * grain
    per_tile = E_pad // NW
    n_chunks = per_tile // CHUNK

    src = edge_index[0].astype(jnp.int32)
    dst = edge_index[1].astype(jnp.int32)
    npadrows = N_PAD - N_NODES
    padv = (N_NODES + jnp.arange(E_pad - E, dtype=jnp.int32) % npadrows)
    src = jnp.concatenate([src, padv])
    dst = jnp.concatenate([dst, padv])

    zeros1 = jnp.zeros((N_PAD,), jnp.float32)
    ones_c = jnp.ones((CHUNK,), jnp.float32)
    xt = jnp.pad(x.T, ((0, 0), (0, N_PAD - N_NODES))).reshape(3, ROWS, LANES)

    deg_fn = pl.kernel(
        functools.partial(_deg_body, n_chunks, per_tile),
        out_type=jax.ShapeDtypeStruct((NC * N_PAD,), jnp.float32),
        mesh=_sc_mesh(),
        scratch_types=[
            pltpu.VMEM((CHUNK,), jnp.int32),
            pltpu.VMEM((CHUNK,), jnp.float32),
            pltpu.VMEM((RPS,), jnp.float32),
            pltpu.VMEM_SHARED((N_PAD,), jnp.float32),
            pltpu.SemaphoreType.DMA,
        ],
    )
    degp = deg_fn(dst, ones_c, zeros1)

    dis_pl, yt_pl = pl.pallas_call(
        _tc_a,
        out_shape=(jax.ShapeDtypeStruct((ROWS, LANES), jnp.float32),
                   jax.ShapeDtypeStruct((3, ROWS, LANES), jnp.float32)),
    )(degp.reshape(NC, ROWS, LANES), xt)

    ytabs = [yt_pl[ch].reshape(N_PAD) for ch in range(3)]

    agg1 = _edge_pass(3, n_chunks, per_tile, src, dst, ytabs, zeros1)
    agg1_pl = [a.reshape(NC, ROWS, LANES) for a in agg1]

    z_pl, q_pl = pl.pallas_call(
        _tc_b,
        out_shape=(jax.ShapeDtypeStruct((ROWS, LANES), jnp.float32),
                   jax.ShapeDtypeStruct((ROWS, LANES), jnp.float32)),
    )(agg1_pl[0], agg1_pl[1], agg1_pl[2], xt, dis_pl, W1, b1, W2)

    (agg2,) = _edge_pass(1, n_chunks, per_tile, src, dst,
                         [z_pl.reshape(N_PAD)], zeros1)

    out_pl = pl.pallas_call(
        _tc_c,
        out_shape=jax.ShapeDtypeStruct((ROWS, LANES), jnp.float32),
    )(agg2.reshape(NC, ROWS, LANES), dis_pl, q_pl, b2)

    return out_pl.reshape(N_PAD)[:N_NODES].reshape(N_NODES, 1)


# speedup vs baseline: 235.5429x; 1.3229x over previous
import functools
import jax
import jax.numpy as jnp
from jax import lax
from jax.experimental import pallas as pl
from jax.experimental.pallas import tpu as pltpu
from jax.experimental.pallas import tpu_sc as plsc

N_NODES = 100000
LANES = 128
ROWS = 800
N_PAD = ROWS * LANES
NC = 2
NS = 16
NW = NC * NS
CHUNK1 = 8192
RPS = N_PAD // NS


def _m8(v):
    return pl.multiple_of(v, 8)


def _sc_mesh():
    return plsc.VectorSubcoreMesh(core_axis_name="c", subcore_axis_name="s")


def _deg_body(n_chunks, per_tile, dst_hbm, ones_hbm, zeros_hbm, out_hbm,
              idx_v, ones_v, stage_v, deg_sh, sem):
    c = lax.axis_index("c")
    s = lax.axis_index("s")
    wid = c * NS + s
    pltpu.sync_copy(zeros_hbm.at[pl.ds(_m8(s * RPS), RPS)], stage_v)
    pltpu.sync_copy(stage_v, deg_sh.at[pl.ds(_m8(s * RPS), RPS)])
    pltpu.sync_copy(ones_hbm, ones_v)
    plsc.subcore_barrier()
    base = wid * per_tile

    def body(i, carry):
        off = _m8(base + i * CHUNK1)
        pltpu.sync_copy(dst_hbm.at[pl.ds(off, CHUNK1)], idx_v)
        pltpu.sync_copy(ones_v, deg_sh.at[idx_v], add=True)
        return carry

    lax.fori_loop(0, n_chunks, body, 0)
    plsc.subcore_barrier()
    pltpu.sync_copy(deg_sh.at[pl.ds(_m8(s * RPS), RPS)], stage_v)
    pltpu.sync_copy(stage_v,
                    out_hbm.at[pl.ds(_m8(c * N_PAD + s * RPS), RPS)])


def _edge_body(nch, n_chunks, per_tile, *refs):
    src_hbm, dst_hbm = refs[0], refs[1]
    tabs_hbm = refs[2:2 + nch]
    zeros_hbm = refs[2 + nch]
    outs_hbm = refs[3 + nch:3 + 2 * nch]
    sc = refs[3 + 2 * nch:]
    isrc = sc[0:2]
    idst = sc[2:4]
    stage_v = sc[4]
    vals = [sc[5:5 + nch], sc[5 + nch:5 + 2 * nch]]
    tabs_sh = sc[5 + 2 * nch:5 + 3 * nch]
    aggs_sh = sc[5 + 3 * nch:5 + 4 * nch]
    sems = sc[5 + 4 * nch:5 + 4 * nch + 2]

    c = lax.axis_index("c")
    s = lax.axis_index("s")
    wid = c * NS + s
    sl = pl.ds(_m8(s * RPS), RPS)
    for k in range(nch):
        pltpu.sync_copy(tabs_hbm[k].at[sl], stage_v)
        pltpu.sync_copy(stage_v, tabs_sh[k].at[sl])
    pltpu.sync_copy(zeros_hbm.at[sl], stage_v)
    for k in range(nch):
        pltpu.sync_copy(stage_v, aggs_sh[k].at[sl])
    plsc.subcore_barrier()
    base = wid * per_tile

    def load_and_fire(i, buf):
        off = _m8(base + i * CHUNK1)
        pltpu.sync_copy(src_hbm.at[pl.ds(off, CHUNK1)], isrc[buf])
        pltpu.sync_copy(dst_hbm.at[pl.ds(off, CHUNK1)], idst[buf])
        for k in range(nch):
            pltpu.async_copy(tabs_sh[k].at[isrc[buf]], vals[buf][k], sems[buf])

    def drain_and_scatter(buf):
        for k in range(nch):
            pltpu.make_async_copy(tabs_sh[k].at[isrc[buf]], vals[buf][k],
                                  sems[buf]).wait()
        for k in range(nch):
            pltpu.sync_copy(vals[buf][k], aggs_sh[k].at[idst[buf]], add=True)

    load_and_fire(0, 0)

    def body(i, carry):
        @pl.when(2 * i + 1 < n_chunks)
        def _():
            load_and_fire(2 * i + 1, 1)

        drain_and_scatter(0)

        @pl.when(2 * i + 2 < n_chunks)
        def _():
            load_and_fire(2 * i + 2, 0)

        @pl.when(2 * i + 1 < n_chunks)
        def _():
            drain_and_scatter(1)

        return carry

    lax.fori_loop(0, (n_chunks + 1) // 2, body, 0)
    plsc.subcore_barrier()
    osl = pl.ds(_m8(c * N_PAD + s * RPS), RPS)
    for k in range(nch):
        pltpu.sync_copy(aggs_sh[k].at[sl], stage_v)
        pltpu.sync_copy(stage_v, outs_hbm[k].at[osl])


def _edge_pass(nch, n_chunks, per_tile, src, dst, tabs, zeros):
    fn = pl.kernel(
        functools.partial(_edge_body, nch, n_chunks, per_tile),
        out_type=tuple(jax.ShapeDtypeStruct((NC * N_PAD,), jnp.float32)
                       for _ in range(nch)),
        mesh=_sc_mesh(),
        scratch_types=(
            [pltpu.VMEM((CHUNK1,), jnp.int32)] * 4
            + [pltpu.VMEM((RPS,), jnp.float32)]
            + [pltpu.VMEM((CHUNK1,), jnp.float32)] * (2 * nch)
            + [pltpu.VMEM_SHARED((N_PAD,), jnp.float32)] * (2 * nch)
            + [pltpu.SemaphoreType.DMA] * 2
        ),
    )
    return fn(src, dst, *tabs, zeros)


def _tc_a(degp_ref, xt_ref, dis_ref, yt_ref):
    deg = degp_ref[0] + degp_ref[1] + 1.0
    dis = lax.rsqrt(deg)
    dis_ref[...] = dis
    for ch in range(3):
        yt_ref[ch] = xt_ref[ch] * dis


def _tc_b(a0_ref, a1_ref, a2_ref, xt_ref, dis_ref, w1_ref, b1_ref, w2_ref,
          z_ref, q_ref):
    dis = dis_ref[...]
    w1 = w1_ref[...]
    b1 = b1_ref[...]
    w2 = w2_ref[...]
    aggs = [a0_ref, a1_ref, a2_ref]
    svec = []
    for ch in range(3):
        agg = aggs[ch][0] + aggs[ch][1]
        svec.append(dis * (agg + dis * xt_ref[ch]))
    q = jnp.zeros_like(dis)
    for j in range(16):
        h = svec[0] * w1[0, j] + svec[1] * w1[1, j] + svec[2] * w1[2, j] + b1[j]
        h = jnp.where(h > 0, h, jnp.exp(h) - 1.0)
        q = q + h * w2[j, 0]
    q_ref[...] = q
    z_ref[...] = q * dis


def _tc_c(aggp_ref, dis_ref, q_ref, b2_ref, out_ref):
    dis = dis_ref[...]
    agg = aggp_ref[0] + aggp_ref[1]
    o = dis * (agg + dis * q_ref[...]) + b2_ref[0]
    out_ref[...] = jnp.where(o > 0, o, jnp.exp(o) - 1.0)


def kernel(x, edge_index, W1, b1, W2, b2):
    E = edge_index.shape[1]
    grain = NW * CHUNK1
    E_pad = ((E + grain - 1) // grain) * grain
    per_tile = E_pad // NW
    n_chunks1 = per_tile // CHUNK1

    src = edge_index[0].astype(jnp.int32)
    dst = edge_index[1].astype(jnp.int32)
    npadrows = N_PAD - N_NODES
    padv = (N_NODES + jnp.arange(E_pad - E, dtype=jnp.int32) % npadrows)
    src = jnp.concatenate([src, padv])
    dst = jnp.concatenate([dst, padv])

    zeros1 = jnp.zeros((N_PAD,), jnp.float32)
    ones_c = jnp.ones((CHUNK1,), jnp.float32)
    xt = jnp.pad(x.T, ((0, 0), (0, N_PAD - N_NODES))).reshape(3, ROWS, LANES)

    deg_fn = pl.kernel(
        functools.partial(_deg_body, n_chunks1, per_tile),
        out_type=jax.ShapeDtypeStruct((NC * N_PAD,), jnp.float32),
        mesh=_sc_mesh(),
        scratch_types=[
            pltpu.VMEM((CHUNK1,), jnp.int32),
            pltpu.VMEM((CHUNK1,), jnp.float32),
            pltpu.VMEM((RPS,), jnp.float32),
            pltpu.VMEM_SHARED((N_PAD,), jnp.float32),
            pltpu.SemaphoreType.DMA,
        ],
    )
    degp = deg_fn(dst, ones_c, zeros1)

    dis_pl, yt_pl = pl.pallas_call(
        _tc_a,
        out_shape=(jax.ShapeDtypeStruct((ROWS, LANES), jnp.float32),
                   jax.ShapeDtypeStruct((3, ROWS, LANES), jnp.float32)),
    )(degp.reshape(NC, ROWS, LANES), xt)

    ytabs = [yt_pl[ch].reshape(N_PAD) for ch in range(3)]

    agg1 = _edge_pass(3, n_chunks1, per_tile, src, dst, ytabs, zeros1)
    agg1_pl = [a.reshape(NC, ROWS, LANES) for a in agg1]

    z_pl, q_pl = pl.pallas_call(
        _tc_b,
        out_shape=(jax.ShapeDtypeStruct((ROWS, LANES), jnp.float32),
                   jax.ShapeDtypeStruct((ROWS, LANES), jnp.float32)),
    )(agg1_pl[0], agg1_pl[1], agg1_pl[2], xt, dis_pl, W1, b1, W2)

    (agg2,) = _edge_pass(1, n_chunks1, per_tile, src, dst,
                         [z_pl.reshape(N_PAD)], zeros1)

    out_pl = pl.pallas_call(
        _tc_c,
        out_shape=jax.ShapeDtypeStruct((ROWS, LANES), jnp.float32),
    )(agg2.reshape(NC, ROWS, LANES), dis_pl, q_pl, b2)

    return out_pl.reshape(N_PAD)[:N_NODES].reshape(N_NODES, 1)


# speedup vs baseline: 244.1503x; 1.0365x over previous
import functools
import jax
import jax.numpy as jnp
from jax import lax
from jax.experimental import pallas as pl
from jax.experimental.pallas import tpu as pltpu
from jax.experimental.pallas import tpu_sc as plsc

N_NODES = 100000
LANES = 128
ROWS = 800
N_PAD = ROWS * LANES
NC = 2
NS = 16
NW = NC * NS
CHUNK1 = 8192
RPS = N_PAD // NS


def _m8(v):
    return pl.multiple_of(v, 8)


def _sc_mesh():
    return plsc.VectorSubcoreMesh(core_axis_name="c", subcore_axis_name="s")


def _deg_body(n_full, tail, dst_hbm, padidx_hbm, ones_hbm, zeros_hbm,
              out_hbm, idx_v, ones_v, stage_v, deg_sh, sem):
    c = lax.axis_index("c")
    s = lax.axis_index("s")
    wid = c * NS + s
    pltpu.sync_copy(zeros_hbm.at[pl.ds(_m8(s * RPS), RPS)], stage_v)
    pltpu.sync_copy(stage_v, deg_sh.at[pl.ds(_m8(s * RPS), RPS)])
    pltpu.sync_copy(ones_hbm, ones_v)
    plsc.subcore_barrier()

    def body(k, carry):
        g = wid + NW * k

        @pl.when(g < n_full)
        def _():
            off = _m8(g * CHUNK1)
            pltpu.sync_copy(dst_hbm.at[pl.ds(off, CHUNK1)], idx_v)
            pltpu.sync_copy(ones_v, deg_sh.at[idx_v], add=True)

        return carry

    lax.fori_loop(0, (n_full + NW - 1) // NW, body, 0)
    if tail:
        @pl.when(wid == NW - 1)
        def _():
            pltpu.sync_copy(padidx_hbm, idx_v)
            pltpu.sync_copy(dst_hbm.at[pl.ds(_m8(n_full * CHUNK1), tail)],
                            idx_v.at[pl.ds(0, tail)])
            pltpu.sync_copy(ones_v, deg_sh.at[idx_v], add=True)
    plsc.subcore_barrier()
    pltpu.sync_copy(deg_sh.at[pl.ds(_m8(s * RPS), RPS)], stage_v)
    pltpu.sync_copy(stage_v,
                    out_hbm.at[pl.ds(_m8(c * N_PAD + s * RPS), RPS)])


def _edge_body(nch, n_full, tail, *refs):
    src_hbm, dst_hbm, padidx_hbm = refs[0], refs[1], refs[2]
    tabs_hbm = refs[3:3 + nch]
    zeros_hbm = refs[3 + nch]
    outs_hbm = refs[4 + nch:4 + 2 * nch]
    sc = refs[4 + 2 * nch:]
    isrc = sc[0:2]
    idst = sc[2:4]
    stage_v = sc[4]
    vals = [sc[5:5 + nch], sc[5 + nch:5 + 2 * nch]]
    tabs_sh = sc[5 + 2 * nch:5 + 3 * nch]
    aggs_sh = sc[5 + 3 * nch:5 + 4 * nch]
    sems = sc[5 + 4 * nch:5 + 4 * nch + 3]

    c = lax.axis_index("c")
    s = lax.axis_index("s")
    wid = c * NS + s
    sl = pl.ds(_m8(s * RPS), RPS)
    for k in range(nch):
        pltpu.sync_copy(tabs_hbm[k].at[sl], stage_v)
        pltpu.sync_copy(stage_v, tabs_sh[k].at[sl])
    pltpu.sync_copy(zeros_hbm.at[sl], stage_v)
    for k in range(nch):
        pltpu.sync_copy(stage_v, aggs_sh[k].at[sl])
    plsc.subcore_barrier()

    def load_and_fire(g, buf):
        off = _m8(g * CHUNK1)
        pltpu.sync_copy(src_hbm.at[pl.ds(off, CHUNK1)], isrc[buf])
        pltpu.sync_copy(dst_hbm.at[pl.ds(off, CHUNK1)], idst[buf])
        for k in range(nch):
            pltpu.async_copy(tabs_sh[k].at[isrc[buf]], vals[buf][k], sems[buf])

    def drain_and_scatter(buf):
        for k in range(nch):
            pltpu.make_async_copy(tabs_sh[k].at[isrc[buf]], vals[buf][k],
                                  sems[buf]).wait()
        cps = [pltpu.async_copy(vals[buf][k], aggs_sh[k].at[idst[buf]],
                                sems[2], add=True) for k in range(nch)]
        for cp in cps:
            cp.wait()

    @pl.when(wid < n_full)
    def _():
        load_and_fire(wid, 0)

    kmax = (n_full + NW - 1) // NW

    def body(i, carry):
        ga = wid + NW * 2 * i
        gb = wid + NW * (2 * i + 1)
        gn = wid + NW * (2 * i + 2)

        @pl.when(gb < n_full)
        def _():
            load_and_fire(gb, 1)

        @pl.when(ga < n_full)
        def _():
            drain_and_scatter(0)

        @pl.when(gn < n_full)
        def _():
            load_and_fire(gn, 0)

        @pl.when(gb < n_full)
        def _():
            drain_and_scatter(1)

        return carry

    lax.fori_loop(0, (kmax + 1) // 2, body, 0)
    if tail:
        @pl.when(wid == NW - 1)
        def _():
            pltpu.sync_copy(padidx_hbm, isrc[0])
            pltpu.sync_copy(padidx_hbm, idst[0])
            toff = _m8(n_full * CHUNK1)
            pltpu.sync_copy(src_hbm.at[pl.ds(toff, tail)],
                            isrc[0].at[pl.ds(0, tail)])
            pltpu.sync_copy(dst_hbm.at[pl.ds(toff, tail)],
                            idst[0].at[pl.ds(0, tail)])
            for k in range(nch):
                pltpu.async_copy(tabs_sh[k].at[isrc[0]], vals[0][k],
                                 sems[0]).wait()
            for k in range(nch):
                pltpu.sync_copy(vals[0][k], aggs_sh[k].at[idst[0]], add=True)
    plsc.subcore_barrier()
    osl = pl.ds(_m8(c * N_PAD + s * RPS), RPS)
    for k in range(nch):
        pltpu.sync_copy(aggs_sh[k].at[sl], stage_v)
        pltpu.sync_copy(stage_v, outs_hbm[k].at[osl])


def _edge_pass(nch, n_full, tail, src, dst, padidx, tabs, zeros):
    fn = pl.kernel(
        functools.partial(_edge_body, nch, n_full, tail),
        out_type=tuple(jax.ShapeDtypeStruct((NC * N_PAD,), jnp.float32)
                       for _ in range(nch)),
        mesh=_sc_mesh(),
        scratch_types=(
            [pltpu.VMEM((CHUNK1,), jnp.int32)] * 4
            + [pltpu.VMEM((RPS,), jnp.float32)]
            + [pltpu.VMEM((CHUNK1,), jnp.float32)] * (2 * nch)
            + [pltpu.VMEM_SHARED((N_PAD,), jnp.float32)] * (2 * nch)
            + [pltpu.SemaphoreType.DMA] * 3
        ),
    )
    return fn(src, dst, padidx, *tabs, zeros)


def _tc_a(degp_ref, xt_ref, dis_ref, yt_ref):
    deg = degp_ref[0] + degp_ref[1] + 1.0
    dis = lax.rsqrt(deg)
    dis_ref[...] = dis
    for ch in range(3):
        yt_ref[ch] = xt_ref[ch] * dis


def _tc_b(a0_ref, a1_ref, a2_ref, xt_ref, dis_ref, w1_ref, b1_ref, w2_ref,
          z_ref, q_ref):
    dis = dis_ref[...]
    w1 = w1_ref[...]
    b1 = b1_ref[...]
    w2 = w2_ref[...]
    aggs = [a0_ref, a1_ref, a2_ref]
    svec = []
    for ch in range(3):
        agg = aggs[ch][0] + aggs[ch][1]
        svec.append(dis * (agg + dis * xt_ref[ch]))
    q = jnp.zeros_like(dis)
    for j in range(16):
        h = svec[0] * w1[0, j] + svec[1] * w1[1, j] + svec[2] * w1[2, j] + b1[j]
        h = jnp.where(h > 0, h, jnp.exp(h) - 1.0)
        q = q + h * w2[j, 0]
    q_ref[...] = q
    z_ref[...] = q * dis


def _tc_c(aggp_ref, dis_ref, q_ref, b2_ref, out_ref):
    dis = dis_ref[...]
    agg = aggp_ref[0] + aggp_ref[1]
    o = dis * (agg + dis * q_ref[...]) + b2_ref[0]
    out_ref[...] = jnp.where(o > 0, o, jnp.exp(o) - 1.0)


def kernel(x, edge_index, W1, b1, W2, b2):
    E = edge_index.shape[1]
    src = edge_index[0].astype(jnp.int32)
    dst = edge_index[1].astype(jnp.int32)
    n_full = E // CHUNK1
    tail = E - n_full * CHUNK1
    if tail % 8 != 0 or n_full < NW:
        pad = (n_full + 1) * CHUNK1 - E if tail else 0
        pad += max(0, NW - (n_full + (1 if tail else 0))) * CHUNK1
        npadrows = N_PAD - N_NODES
        padv = N_NODES + jnp.arange(pad, dtype=jnp.int32) % npadrows
        src = jnp.concatenate([src, padv])
        dst = jnp.concatenate([dst, padv])
        n_full = (E + pad) // CHUNK1
        tail = 0

    npadrows = N_PAD - N_NODES
    padidx = (N_NODES
              + jnp.arange(CHUNK1, dtype=jnp.int32) % npadrows)
    zeros1 = jnp.zeros((N_PAD,), jnp.float32)
    ones_c = jnp.ones((CHUNK1,), jnp.float32)
    xt = jnp.pad(x.T, ((0, 0), (0, N_PAD - N_NODES))).reshape(3, ROWS, LANES)

    deg_fn = pl.kernel(
        functools.partial(_deg_body, n_full, tail),
        out_type=jax.ShapeDtypeStruct((NC * N_PAD,), jnp.float32),
        mesh=_sc_mesh(),
        scratch_types=[
            pltpu.VMEM((CHUNK1,), jnp.int32),
            pltpu.VMEM((CHUNK1,), jnp.float32),
            pltpu.VMEM((RPS,), jnp.float32),
            pltpu.VMEM_SHARED((N_PAD,), jnp.float32),
            pltpu.SemaphoreType.DMA,
        ],
    )
    degp = deg_fn(dst, padidx, ones_c, zeros1)

    dis_pl, yt_pl = pl.pallas_call(
        _tc_a,
        out_shape=(jax.ShapeDtypeStruct((ROWS, LANES), jnp.float32),
                   jax.ShapeDtypeStruct((3, ROWS, LANES), jnp.float32)),
    )(degp.reshape(NC, ROWS, LANES), xt)

    ytabs = [yt_pl[ch].reshape(N_PAD) for ch in range(3)]

    agg1 = _edge_pass(3, n_full, tail, src, dst, padidx, ytabs, zeros1)
    agg1_pl = [a.reshape(NC, ROWS, LANES) for a in agg1]

    z_pl, q_pl = pl.pallas_call(
        _tc_b,
        out_shape=(jax.ShapeDtypeStruct((ROWS, LANES), jnp.float32),
                   jax.ShapeDtypeStruct((ROWS, LANES), jnp.float32)),
    )(agg1_pl[0], agg1_pl[1], agg1_pl[2], xt, dis_pl, W1, b1, W2)

    (agg2,) = _edge_pass(1, n_full, tail, src, dst, padidx,
                         [z_pl.reshape(N_PAD)], zeros1)

    out_pl = pl.pallas_call(
        _tc_c,
        out_shape=jax.ShapeDtypeStruct((ROWS, LANES), jnp.float32),
    )(agg2.reshape(NC, ROWS, LANES), dis_pl, q_pl, b2)

    return out_pl.reshape(N_PAD)[:N_NODES].reshape(N_NODES, 1)
